# baseline (speedup 1.0000x reference)
import functools

import jax
import jax.numpy as jnp
from jax import lax
from jax.experimental import pallas as pl
from jax.experimental.pallas import tpu as pltpu
from jax.experimental.pallas import tpu_sc as plsc

NUM_CORES = 2
NUM_SUBCORES = 16
NW = NUM_CORES * NUM_SUBCORES

BATCH = 4096
HIST = 200
DIM = 64
NV = 1000000
BBLK = BATCH // NW
LANES = 16
HB = 1
SLOTS = HB * BBLK
NBLK = HIST // HB
OBP = BBLK + 5


@functools.partial(
    pl.kernel,
    out_type=jax.ShapeDtypeStruct((HIST, DIM, BATCH), jnp.float32),
    mesh=plsc.VectorSubcoreMesh(core_axis_name="c", subcore_axis_name="s"),
    scratch_types=(
        [pltpu.VMEM((HIST, BBLK), jnp.int32)]
        + [pltpu.VMEM((SLOTS,), jnp.int32) for _ in range(2)]
        + [pltpu.VMEM((SLOTS, 2 * DIM), jnp.float32) for _ in range(2)]
        + [pltpu.VMEM((HB, DIM, OBP), jnp.float32) for _ in range(2)]
        + [pltpu.SemaphoreType.DMA for _ in range(4)]
    ),
    compiler_params=pltpu.CompilerParams(needs_layout_passes=False),
)
def _gather_kernel(xt_hbm, t2_hbm, out_hbm, idx_all, ip0, ip1,
                   pr0, pr1, ob0, ob1, sg0, sg1, sw0, sw1):
    ipair = [ip0, ip1]
    pair = [pr0, pr1]
    obuf = [ob0, ob1]
    sg = [sg0, sg1]
    sw = [sw0, sw1]

    wid = lax.axis_index("s") * NUM_CORES + lax.axis_index("c")
    bbase = wid * BBLK
    pltpu.sync_copy(xt_hbm.at[:, pl.ds(bbase, BBLK)], idx_all)

    iota = lax.iota(jnp.int32, LANES)

    def prep_indices(j, b):
        h0 = HB * j
        for hl in range(HB):
            for g in range(BBLK // LANES):
                v = idx_all[h0 + hl, pl.ds(LANES * g, LANES)]
                ipair[b][pl.ds(hl * BBLK + LANES * g, LANES)] = (
                    lax.shift_right_logical(v, 1)
                )

    def issue_gather(b):
        pltpu.async_copy(t2_hbm.at[ipair[b]], pair[b], sg[b])

    def wait_gather(b):
        pltpu.make_async_copy(t2_hbm.at[ipair[b]], pair[b], sg[b]).wait()

    def issue_write(j, b):
        pltpu.async_copy(
            obuf[b].at[:, :, pl.ds(0, BBLK)],
            out_hbm.at[pl.ds(HB * j, HB), :, pl.ds(bbase, BBLK)],
            sw[b],
        )

    def wait_write(b):
        pltpu.make_async_copy(
            obuf[b].at[:, :, pl.ds(0, BBLK)],
            out_hbm.at[pl.ds(0, HB), :, pl.ds(0, BBLK)],
            sw[b],
        ).wait()

    def transpose(j, b):
        pr = pair[b]
        ob = obuf[b]
        h0 = HB * j
        rowms = [iota + (LANES * m) for m in range(2 * DIM // LANES)]

        for hl in range(HB):
            hrow = jnp.full((LANES,), h0 + hl, jnp.int32)

            @plsc.parallel_loop(0, BBLK, LANES, unroll=2)
            def sgbody(s0):
                s0v = jnp.full((LANES,), s0, jnp.int32)
                for i in range(LANES):
                    bvec = s0v + i
                    hv = (plsc.load_gather(idx_all, [hrow, bvec]) & 1) * DIM
                    ok_lo = hv < DIM
                    ok_hi = hv >= DIM
                    for m in range(2 * DIM // LANES):
                        v = pr[(hl * BBLK) + s0 + i, pl.ds(LANES * m, LANES)]
                        ok = ok_lo if m < DIM // LANES else ok_hi
                        plsc.store_scatter(
                            ob.at[hl], [rowms[m] - hv, bvec], v, mask=ok
                        )

    prep_indices(0, 0)
    issue_gather(0)

    def jbody(j2, _):
        j = 2 * j2

        prep_indices(j + 1, 1)
        issue_gather(1)

        @pl.when(j2 > 0)
        def _():
            wait_write(0)

        wait_gather(0)
        transpose(j, 0)
        issue_write(j, 0)

        @pl.when(j + 2 < NBLK)
        def _():
            prep_indices(j + 2, 0)
            issue_gather(0)

        @pl.when(j2 > 0)
        def _():
            wait_write(1)

        wait_gather(1)
        transpose(j + 1, 1)
        issue_write(j + 1, 1)
        return ()

    lax.fori_loop(0, NBLK // 2, jbody, (), unroll=False)
    wait_write(0)
    wait_write(1)


def kernel(x, table):
    t2 = table.reshape(NV // 2, 2 * DIM)
    out2 = _gather_kernel(x.T, t2)
    return out2.transpose(2, 0, 1)


# speedup vs baseline: 1.4004x; 1.4004x over previous
import functools

import jax
import jax.numpy as jnp
from jax import lax
from jax.experimental import pallas as pl
from jax.experimental.pallas import tpu as pltpu
from jax.experimental.pallas import tpu_sc as plsc

NUM_CORES = 2
NUM_SUBCORES = 16
NW = NUM_CORES * NUM_SUBCORES

BATCH = 4096
HIST = 200
DIM = 64
TOTAL = BATCH * HIST
B_PER_W = TOTAL // NW
CHUNK = 512
NBUF = 2
GROUP = CHUNK * NBUF
N_GROUP = B_PER_W // GROUP


@functools.partial(
    pl.kernel,
    out_type=jax.ShapeDtypeStruct((TOTAL, DIM), jnp.float32),
    mesh=plsc.VectorSubcoreMesh(core_axis_name="c", subcore_axis_name="s"),
    scratch_types=(
        [pltpu.VMEM((B_PER_W,), jnp.int32)]
        + [pltpu.VMEM((CHUNK, DIM), jnp.float32) for _ in range(NBUF)]
        + [pltpu.SemaphoreType.DMA for _ in range(2 * NBUF)]
    ),
    compiler_params=pltpu.CompilerParams(use_tc_tiling_on_sc=False),
)
def _gather_kernel(idx_hbm, table_hbm, out_hbm, idx_v, *bufs_and_sems):
    rows = list(bufs_and_sems[:NBUF])
    sem_g = list(bufs_and_sems[NBUF:2 * NBUF])
    sem_w = list(bufs_and_sems[2 * NBUF:])

    wid = lax.axis_index("s") * NUM_CORES + lax.axis_index("c")
    base = wid * B_PER_W
    pltpu.sync_copy(idx_hbm.at[pl.ds(base, B_PER_W)], idx_v)

    def start_gather(chunk, b):
        idx_slice = idx_v.at[pl.ds(chunk * CHUNK, CHUNK)]
        pltpu.async_copy(table_hbm.at[idx_slice], rows[b], sem_g[b])

    def start_write(chunk, b):
        pltpu.async_copy(
            rows[b], out_hbm.at[pl.ds(base + chunk * CHUNK, CHUNK)], sem_w[b]
        )

    def wait_gather(b):
        pltpu.make_async_copy(
            table_hbm.at[idx_v.at[pl.ds(0, CHUNK)]], rows[b], sem_g[b]
        ).wait()

    def wait_write(b):
        pltpu.make_async_copy(
            rows[b], out_hbm.at[pl.ds(0, CHUNK)], sem_w[b]
        ).wait()

    def group_body(gi, _):
        for b in range(NBUF):
            @pl.when(gi > 0)
            def _():
                wait_write(b)
            start_gather(gi * NBUF + b, b)
        for b in range(NBUF):
            wait_gather(b)
            start_write(gi * NBUF + b, b)
        return ()

    lax.fori_loop(0, N_GROUP, group_body, (), unroll=False)
    for b in range(NBUF):
        wait_write(b)


def kernel(x, table):
    idx = x.reshape(TOTAL).astype(jnp.int32)
    out = _gather_kernel(idx, table)
    return out.reshape(BATCH, HIST, DIM)
